# baseline (speedup 1.0000x reference)
import functools

import jax
import jax.numpy as jnp
from jax import lax
from jax.experimental import pallas as pl
from jax.experimental.pallas import tpu as pltpu
from jax.experimental.pallas import tpu_sc as plsc

N = 10000
D = 128
NW = 32
RPW = 320
NPAD = NW * RPW
BLK = 256
NBLK = NPAD // BLK
ECHUNK = 2048
GCH = 128
PCAP = 2 * ECHUNK + 16
SHIFT = 14
DMASK = (1 << SHIFT) - 1
EPS = 1e-5

_mesh = plsc.VectorSubcoreMesh(core_axis_name="c", subcore_axis_name="s")


def _tc_pre_body(x_ref, wp_ref, bp_ref, ws_ref, m_ref, s_ref):
    x = x_ref[...]
    m_ref[...] = jnp.maximum(
        jnp.dot(x, wp_ref[...], preferred_element_type=jnp.float32)
        + bp_ref[...], 0.0).astype(jnp.bfloat16)
    s_ref[...] = jnp.dot(x, ws_ref[...], preferred_element_type=jnp.float32)


def _tc_pre(x, wp, bp, ws):
    return pl.pallas_call(
        _tc_pre_body,
        grid=(NBLK,),
        in_specs=[
            pl.BlockSpec((BLK, D), lambda i: (i, 0)),
            pl.BlockSpec((D, D), lambda i: (0, 0)),
            pl.BlockSpec((1, D), lambda i: (0, 0)),
            pl.BlockSpec((D, D), lambda i: (0, 0)),
        ],
        out_specs=[
            pl.BlockSpec((BLK, D), lambda i: (i, 0)),
            pl.BlockSpec((BLK, D), lambda i: (i, 0)),
        ],
        out_shape=[
            jax.ShapeDtypeStruct((NPAD, D), jnp.bfloat16),
            jax.ShapeDtypeStruct((NPAD, D), jnp.float32),
        ],
    )(x, wp, bp, ws)


def _tc_combine_body(s_ref, a_ref, wn_ref, b_ref, y_ref, sums_ref, acc):
    i = pl.program_id(0)
    y = (s_ref[...]
         + jnp.dot(a_ref[...].astype(jnp.float32), wn_ref[...],
                   preferred_element_type=jnp.float32)
         + b_ref[...])
    y_ref[...] = y

    @pl.when(i == 0)
    def _():
        acc[...] = jnp.zeros_like(acc)

    rid = i * BLK + lax.broadcasted_iota(jnp.int32, (BLK, 1), 0)
    ym = y * (rid < N).astype(jnp.float32)
    acc[0:1, :] = acc[0:1, :] + jnp.sum(ym, axis=0, keepdims=True)
    acc[1:2, :] = acc[1:2, :] + jnp.sum(ym * y, axis=0, keepdims=True)

    @pl.when(i == NBLK - 1)
    def _():
        sums_ref[...] = acc[...]


def _tc_combine(s, a, wn, b):
    return pl.pallas_call(
        _tc_combine_body,
        grid=(NBLK,),
        in_specs=[
            pl.BlockSpec((BLK, D), lambda i: (i, 0)),
            pl.BlockSpec((BLK, D), lambda i: (i, 0)),
            pl.BlockSpec((D, D), lambda i: (0, 0)),
            pl.BlockSpec((1, D), lambda i: (0, 0)),
        ],
        out_specs=[
            pl.BlockSpec((BLK, D), lambda i: (i, 0)),
            pl.BlockSpec((8, D), lambda i: (0, 0)),
        ],
        out_shape=[
            jax.ShapeDtypeStruct((NPAD, D), jnp.float32),
            jax.ShapeDtypeStruct((8, D), jnp.float32),
        ],
        scratch_shapes=[pltpu.VMEM((8, D), jnp.float32)],
        compiler_params=pltpu.CompilerParams(
            dimension_semantics=("arbitrary",)),
    )(s, a, wn, b)


def _bn_from_sums(sums):
    mean = sums[0:1, :] * (1.0 / N)
    var = sums[1:2, :] * (1.0 / N) - mean * mean
    inv = lax.rsqrt(var + EPS)
    return mean, inv


def _tc_bn_next_body(y_ref, sums_ref, g_ref, be_ref, wp_ref, bp_ref, ws_ref,
                     m_ref, s_ref):
    mean, inv = _bn_from_sums(sums_ref[...])
    h = jnp.maximum((y_ref[...] - mean) * (inv * g_ref[...]) + be_ref[...],
                    0.0)
    m_ref[...] = jnp.maximum(
        jnp.dot(h, wp_ref[...], preferred_element_type=jnp.float32)
        + bp_ref[...], 0.0).astype(jnp.bfloat16)
    s_ref[...] = jnp.dot(h, ws_ref[...], preferred_element_type=jnp.float32)


def _tc_bn_next(y, sums, g, be, wp, bp, ws):
    return pl.pallas_call(
        _tc_bn_next_body,
        grid=(NBLK,),
        in_specs=[
            pl.BlockSpec((BLK, D), lambda i: (i, 0)),
            pl.BlockSpec((8, D), lambda i: (0, 0)),
            pl.BlockSpec((1, D), lambda i: (0, 0)),
            pl.BlockSpec((1, D), lambda i: (0, 0)),
            pl.BlockSpec((D, D), lambda i: (0, 0)),
            pl.BlockSpec((1, D), lambda i: (0, 0)),
            pl.BlockSpec((D, D), lambda i: (0, 0)),
        ],
        out_specs=[
            pl.BlockSpec((BLK, D), lambda i: (i, 0)),
            pl.BlockSpec((BLK, D), lambda i: (i, 0)),
        ],
        out_shape=[
            jax.ShapeDtypeStruct((NPAD, D), jnp.bfloat16),
            jax.ShapeDtypeStruct((NPAD, D), jnp.float32),
        ],
    )(y, sums, g, be, wp, bp, ws)


def _tc_bn_final_body(y_ref, sums_ref, g_ref, be_ref, o_ref):
    mean, inv = _bn_from_sums(sums_ref[...])
    o_ref[...] = jnp.maximum(
        (y_ref[...] - mean) * (inv * g_ref[...]) + be_ref[...], 0.0)


def _tc_bn_final(y, sums, g, be):
    return pl.pallas_call(
        _tc_bn_final_body,
        grid=(NBLK,),
        in_specs=[
            pl.BlockSpec((BLK, D), lambda i: (i, 0)),
            pl.BlockSpec((8, D), lambda i: (0, 0)),
            pl.BlockSpec((1, D), lambda i: (0, 0)),
            pl.BlockSpec((1, D), lambda i: (0, 0)),
        ],
        out_specs=pl.BlockSpec((BLK, D), lambda i: (i, 0)),
        out_shape=jax.ShapeDtypeStruct((NPAD, D), jnp.float32),
    )(y, sums, g, be)


def _worker(lo_only=False):
    w = lax.axis_index("c") * 16 + lax.axis_index("s")
    lo = w * RPW
    return (lo,) if lo_only else (w, lo)


def _make_partition(epad, cap, npairs):
    nch = 2 * npairs

    @functools.partial(
        pl.kernel,
        out_type=(jax.ShapeDtypeStruct((NW * cap,), jnp.int32),
                  jax.ShapeDtypeStruct((NW * 16,), jnp.int32)),
        mesh=_mesh,
        compiler_params=pltpu.CompilerParams(needs_layout_passes=False),
        scratch_types=[
            pltpu.VMEM((ECHUNK,), jnp.int32),
            pltpu.VMEM((ECHUNK,), jnp.int32),
            pltpu.VMEM((PCAP,), jnp.int32),
            pltpu.VMEM((16,), jnp.int32),
            pltpu.SemaphoreType.DMA,
            pltpu.SemaphoreType.DMA,
        ],
    )
    def part(code_hbm, list_hbm, counts_hbm, buf0, buf1, pend, cbuf,
             sem0, sem1):
        w, lo = _worker()
        lbase = w * cap
        dummy_code = lo + RPW

        pltpu.async_copy(code_hbm.at[pl.ds(0, ECHUNK)], buf0, sem0)
        pltpu.async_copy(code_hbm.at[pl.ds(ECHUNK, ECHUNK)], buf1, sem1)

        def scan_chunk(buf, cnt0):
            def ibody(j, cnt):
                v = buf[pl.ds(j * 16, 16)]
                du = (v & DMASK) - lo
                mask = plsc.bitcast(du, jnp.uint32) < jnp.uint32(RPW)
                csum = plsc.cumsum(mask.astype(jnp.int32))
                plsc.store_scatter(pend, [cnt + csum - 1], v, mask=mask)
                return cnt + plsc.all_reduce_population_count(mask)[0]
            return lax.fori_loop(0, ECHUNK // 16, ibody, cnt0)

        def flush(cnt, optr):
            do = cnt >= ECHUNK

            @pl.when(do)
            def _():
                pltpu.sync_copy(
                    pend.at[pl.ds(0, ECHUNK)],
                    list_hbm.at[pl.ds(pl.multiple_of(lbase + optr, 8),
                                      ECHUNK)])

                def mv(t, _):
                    pend[pl.ds(t * 16, 16)] = pend[pl.ds(ECHUNK + t * 16, 16)]
                    return 0
                lax.fori_loop(0, (PCAP - ECHUNK) // 16, mv, 0)

            cnt = jnp.where(do, cnt - ECHUNK, cnt)
            optr = jnp.where(do, optr + ECHUNK, optr)
            return cnt, optr

        def pbody(p, carry):
            cnt, optr = carry
            pltpu.make_async_copy(code_hbm.at[pl.ds(0, ECHUNK)], buf0,
                                  sem0).wait()
            cnt = scan_chunk(buf0, cnt)
            cnt, optr = flush(cnt, optr)

            @pl.when(2 * p + 2 < nch)
            def _():
                pltpu.async_copy(
                    code_hbm.at[pl.ds((2 * p + 2) * ECHUNK, ECHUNK)],
                    buf0, sem0)

            pltpu.make_async_copy(code_hbm.at[pl.ds(0, ECHUNK)], buf1,
                                  sem1).wait()
            cnt = scan_chunk(buf1, cnt)
            cnt, optr = flush(cnt, optr)

            @pl.when(2 * p + 3 < nch)
            def _():
                pltpu.async_copy(
                    code_hbm.at[pl.ds((2 * p + 3) * ECHUNK, ECHUNK)],
                    buf1, sem1)

            return cnt, optr

        cnt, optr = lax.fori_loop(0, npairs, pbody, (jnp.int32(0),
                                                     jnp.int32(0)))

        dvec = jnp.full((16,), dummy_code, jnp.int32)
        for k in range(8):
            pend[pl.ds(cnt + k * 16, 16)] = dvec
        cnt_pad = ((cnt + 127) // 128) * 128
        pltpu.sync_copy(
            pend.at[pl.ds(0, ECHUNK)],
            list_hbm.at[pl.ds(pl.multiple_of(lbase + optr, 8), ECHUNK)])
        pltpu.sync_copy(
            pend.at[pl.ds(ECHUNK, ECHUNK)],
            list_hbm.at[pl.ds(pl.multiple_of(lbase + optr + ECHUNK, 8),
                              ECHUNK)])
        cbuf[...] = jnp.full((16,), optr + cnt_pad, jnp.int32)
        pltpu.sync_copy(cbuf,
                        counts_hbm.at[pl.ds(pl.multiple_of(w * 16, 8), 16)])

    return part


def _make_segmax(cap):
    @functools.partial(
        pl.kernel,
        out_type=jax.ShapeDtypeStruct((NPAD, D // 2), jnp.int32),
        mesh=_mesh,
        compiler_params=pltpu.CompilerParams(needs_layout_passes=False,
                                             use_tc_tiling_on_sc=False),
        scratch_types=[
            pltpu.VMEM((RPW + 1, 16), jnp.int32),
            pltpu.VMEM((RPW + 1, 16), jnp.int32),
            pltpu.VMEM((RPW + 1, 16), jnp.int32),
            pltpu.VMEM((RPW + 1, 16), jnp.int32),
            pltpu.VMEM((GCH,), jnp.int32),
            pltpu.VMEM((GCH,), jnp.int32),
            pltpu.VMEM((GCH,), jnp.int32),
            pltpu.VMEM((GCH,), jnp.int32),
            pltpu.VMEM((GCH,), jnp.int32),
            pltpu.VMEM((GCH,), jnp.int32),
            pltpu.VMEM((GCH, D // 2), jnp.int32),
            pltpu.VMEM((GCH, D // 2), jnp.int32),
            pltpu.VMEM((16,), jnp.int32),
            pltpu.SemaphoreType.DMA,
            pltpu.SemaphoreType.DMA,
            pltpu.SemaphoreType.DMA,
            pltpu.SemaphoreType.DMA,
        ],
    )
    def segmax(m_hbm, list_hbm, counts_hbm, agg_hbm, acc0, acc1, acc2, acc3,
               codesA, codesB, sbufA, sbufB, dbufA, dbufB, rowsA, rowsB,
               cbuf, semA, semB, semcA, semcB):
        w, lo = _worker()
        lbase = w * cap
        accs = (acc0, acc1, acc2, acc3)
        z = jnp.zeros((16,), jnp.int32)
        for c, a in enumerate(accs):
            pltpu.sync_copy(
                m_hbm.at[pl.ds(pl.multiple_of(lo, 8), RPW),
                         pl.ds(c * 16, 16)],
                a.at[pl.ds(0, RPW)])
            a[RPW, :] = z
        pltpu.sync_copy(counts_hbm.at[pl.ds(pl.multiple_of(w * 16, 8), 16)],
                        cbuf)
        nch = cbuf[...][0] // GCH

        def codes_at(g):
            return list_hbm.at[pl.ds(pl.multiple_of(lbase + g * GCH, 8),
                                     GCH)]

        def decode(codes, sbuf, dbuf):
            for k in range(GCH // 16):
                sl = pl.ds(k * 16, 16)
                v = codes[sl]
                sbuf[sl] = lax.shift_right_logical(v, SHIFT)
                dbuf[sl] = (v & DMASK) - lo

        def accumulate(dbuf, rows):
            def ebody(k, _):
                dvec = dbuf[pl.ds(k * 16, 16)]
                for lane in range(16):
                    d = dvec[lane]
                    i = k * 16 + lane
                    for c, ac in enumerate(accs):
                        a = plsc.bitcast(ac[d, :], jnp.bfloat16)
                        r = plsc.bitcast(rows[i, pl.ds(c * 16, 16)],
                                         jnp.bfloat16)
                        ac[d, :] = plsc.bitcast(jnp.maximum(a, r),
                                                jnp.int32)
                return 0
            lax.fori_loop(0, GCH // 16, ebody, 0)

        @pl.when(nch > 0)
        def _():
            pltpu.sync_copy(codes_at(0), codesA)
            decode(codesA, sbufA, dbufA)
            pltpu.async_copy(m_hbm.at[sbufA], rowsA, semA)

            @pl.when(nch > 1)
            def _():
                pltpu.async_copy(codes_at(1), codesB, semcB)

            def pbody(p, _):
                g = 2 * p

                @pl.when(g + 1 < nch)
                def _():
                    pltpu.make_async_copy(codes_at(1), codesB, semcB).wait()
                    decode(codesB, sbufB, dbufB)
                    pltpu.async_copy(m_hbm.at[sbufB], rowsB, semB)

                @pl.when(g + 2 < nch)
                def _():
                    pltpu.async_copy(codes_at(g + 2), codesA, semcA)

                pltpu.make_async_copy(m_hbm.at[sbufA], rowsA, semA).wait()
                accumulate(dbufA, rowsA)

                @pl.when(g + 2 < nch)
                def _():
                    pltpu.make_async_copy(codes_at(0), codesA, semcA).wait()
                    decode(codesA, sbufA, dbufA)
                    pltpu.async_copy(m_hbm.at[sbufA], rowsA, semA)

                @pl.when(g + 3 < nch)
                def _():
                    pltpu.async_copy(codes_at(g + 3), codesB, semcB)

                @pl.when(g + 1 < nch)
                def _():
                    pltpu.make_async_copy(m_hbm.at[sbufB], rowsB, semB).wait()
                    accumulate(dbufB, rowsB)
                return 0

            lax.fori_loop(0, (nch + 1) // 2, pbody, 0)

        for c, a in enumerate(accs):
            pltpu.sync_copy(
                a.at[pl.ds(0, RPW)],
                agg_hbm.at[pl.ds(pl.multiple_of(lo, 8), RPW),
                           pl.ds(c * 16, 16)])

    return segmax


def kernel(feat, edge_index, W_pool1, b_pool1, W_self1, W_neigh1, b1,
           gamma1, beta1, W_pool2, b_pool2, W_self2, W_neigh2, b2,
           gamma2, beta2):
    n, d = feat.shape
    assert n == N and d == D
    e = edge_index.shape[1]
    npairs = (e + 2 * ECHUNK - 1) // (2 * ECHUNK)
    epad = npairs * 2 * ECHUNK
    cap = epad + 2 * ECHUNK

    feat_p = jnp.pad(feat, ((0, NPAD - n), (0, 0)))
    code = (edge_index[0].astype(jnp.int32) << SHIFT) \
        | edge_index[1].astype(jnp.int32)
    code_p = jnp.pad(code, (0, epad - e), constant_values=DMASK)

    r = lambda v: v.reshape(1, D)
    pack = lambda m: lax.bitcast_convert_type(
        m.reshape(NPAD, D // 2, 2), jnp.int32)
    unpack = lambda a32: lax.bitcast_convert_type(
        a32, jnp.bfloat16).reshape(NPAD, D)

    elist, counts = _make_partition(epad, cap, npairs)(code_p)

    m1, s1 = _tc_pre(feat_p, W_pool1, r(b_pool1), W_self1)
    agg1 = _make_segmax(cap)(pack(m1), elist, counts)
    y1, sums1 = _tc_combine(s1, unpack(agg1), W_neigh1, r(b1))
    m2, s2 = _tc_bn_next(y1, sums1, r(gamma1), r(beta1), W_pool2,
                         r(b_pool2), W_self2)
    agg2 = _make_segmax(cap)(pack(m2), elist, counts)
    y2, sums2 = _tc_combine(s2, unpack(agg2), W_neigh2, r(b2))
    out = _tc_bn_final(y2, sums2, r(gamma2), r(beta2))
    return out[:n]


# speedup vs baseline: 1.1076x; 1.1076x over previous
import functools

import jax
import jax.numpy as jnp
from jax import lax
from jax.experimental import pallas as pl
from jax.experimental.pallas import tpu as pltpu
from jax.experimental.pallas import tpu_sc as plsc

N = 10000
D = 128
NW = 32
RPW = 320
NPAD = NW * RPW
BLK = 256
NBLK = NPAD // BLK
ECHUNK = 2048
GCH = 128
PCAP = 2 * ECHUNK + 16
SHIFT = 14
DMASK = (1 << SHIFT) - 1
EPS = 1e-5

_mesh = plsc.VectorSubcoreMesh(core_axis_name="c", subcore_axis_name="s")


def _rne16(v):
    i = lax.bitcast_convert_type(v, jnp.int32)
    lsb = lax.shift_right_logical(i, 16) & 1
    return (i + 0x7FFF + lsb) & jnp.int32(-65536)


def _pack_bf16(even_f32, odd_f32):
    return _rne16(odd_f32) | lax.shift_right_logical(_rne16(even_f32), 16)


def _unpack_bf16(a32):
    even = lax.bitcast_convert_type(a32 << 16, jnp.float32)
    odd = lax.bitcast_convert_type(a32 & jnp.int32(-65536), jnp.float32)
    return even, odd


def _mm(x, w_ref):
    return jnp.dot(x, w_ref[...], preferred_element_type=jnp.float32)


def _tc_pre_body(x_ref, wpe_ref, wpo_ref, bpe_ref, bpo_ref, ws_ref,
                 m_ref, s_ref):
    x = x_ref[...]
    me = jnp.maximum(_mm(x, wpe_ref) + bpe_ref[...], 0.0)
    mo = jnp.maximum(_mm(x, wpo_ref) + bpo_ref[...], 0.0)
    m_ref[...] = _pack_bf16(me, mo)
    s_ref[...] = _mm(x, ws_ref)


def _tc_pre(x, wpe, wpo, bpe, bpo, ws):
    return pl.pallas_call(
        _tc_pre_body,
        grid=(NBLK,),
        in_specs=[
            pl.BlockSpec((BLK, D), lambda i: (i, 0)),
            pl.BlockSpec((D, D // 2), lambda i: (0, 0)),
            pl.BlockSpec((D, D // 2), lambda i: (0, 0)),
            pl.BlockSpec((1, D // 2), lambda i: (0, 0)),
            pl.BlockSpec((1, D // 2), lambda i: (0, 0)),
            pl.BlockSpec((D, D), lambda i: (0, 0)),
        ],
        out_specs=[
            pl.BlockSpec((BLK, D // 2), lambda i: (i, 0)),
            pl.BlockSpec((BLK, D), lambda i: (i, 0)),
        ],
        out_shape=[
            jax.ShapeDtypeStruct((NPAD, D // 2), jnp.int32),
            jax.ShapeDtypeStruct((NPAD, D), jnp.float32),
        ],
    )(x, wpe, wpo, bpe, bpo, ws)


def _tc_combine_body(s_ref, a_ref, wne_ref, wno_ref, b_ref, y_ref,
                     sums_ref, acc):
    i = pl.program_id(0)
    ae, ao = _unpack_bf16(a_ref[...])
    y = (s_ref[...] + _mm(ae, wne_ref) + _mm(ao, wno_ref) + b_ref[...])
    y_ref[...] = y

    @pl.when(i == 0)
    def _():
        acc[...] = jnp.zeros_like(acc)

    rid = i * BLK + lax.broadcasted_iota(jnp.int32, (BLK, 1), 0)
    ym = y * (rid < N).astype(jnp.float32)
    acc[0:1, :] = acc[0:1, :] + jnp.sum(ym, axis=0, keepdims=True)
    acc[1:2, :] = acc[1:2, :] + jnp.sum(ym * y, axis=0, keepdims=True)

    @pl.when(i == NBLK - 1)
    def _():
        sums_ref[...] = acc[...]


def _tc_combine(s, a, wne, wno, b):
    return pl.pallas_call(
        _tc_combine_body,
        grid=(NBLK,),
        in_specs=[
            pl.BlockSpec((BLK, D), lambda i: (i, 0)),
            pl.BlockSpec((BLK, D // 2), lambda i: (i, 0)),
            pl.BlockSpec((D // 2, D), lambda i: (0, 0)),
            pl.BlockSpec((D // 2, D), lambda i: (0, 0)),
            pl.BlockSpec((1, D), lambda i: (0, 0)),
        ],
        out_specs=[
            pl.BlockSpec((BLK, D), lambda i: (i, 0)),
            pl.BlockSpec((8, D), lambda i: (0, 0)),
        ],
        out_shape=[
            jax.ShapeDtypeStruct((NPAD, D), jnp.float32),
            jax.ShapeDtypeStruct((8, D), jnp.float32),
        ],
        scratch_shapes=[pltpu.VMEM((8, D), jnp.float32)],
        compiler_params=pltpu.CompilerParams(
            dimension_semantics=("arbitrary",)),
    )(s, a, wne, wno, b)


def _bn_from_sums(sums):
    mean = sums[0:1, :] * (1.0 / N)
    var = sums[1:2, :] * (1.0 / N) - mean * mean
    inv = lax.rsqrt(var + EPS)
    return mean, inv


def _tc_bn_next_body(y_ref, sums_ref, g_ref, be_ref, wpe_ref, wpo_ref,
                     bpe_ref, bpo_ref, ws_ref, m_ref, s_ref):
    mean, inv = _bn_from_sums(sums_ref[...])
    h = jnp.maximum((y_ref[...] - mean) * (inv * g_ref[...]) + be_ref[...],
                    0.0)
    me = jnp.maximum(_mm(h, wpe_ref) + bpe_ref[...], 0.0)
    mo = jnp.maximum(_mm(h, wpo_ref) + bpo_ref[...], 0.0)
    m_ref[...] = _pack_bf16(me, mo)
    s_ref[...] = _mm(h, ws_ref)


def _tc_bn_next(y, sums, g, be, wpe, wpo, bpe, bpo, ws):
    return pl.pallas_call(
        _tc_bn_next_body,
        grid=(NBLK,),
        in_specs=[
            pl.BlockSpec((BLK, D), lambda i: (i, 0)),
            pl.BlockSpec((8, D), lambda i: (0, 0)),
            pl.BlockSpec((1, D), lambda i: (0, 0)),
            pl.BlockSpec((1, D), lambda i: (0, 0)),
            pl.BlockSpec((D, D // 2), lambda i: (0, 0)),
            pl.BlockSpec((D, D // 2), lambda i: (0, 0)),
            pl.BlockSpec((1, D // 2), lambda i: (0, 0)),
            pl.BlockSpec((1, D // 2), lambda i: (0, 0)),
            pl.BlockSpec((D, D), lambda i: (0, 0)),
        ],
        out_specs=[
            pl.BlockSpec((BLK, D // 2), lambda i: (i, 0)),
            pl.BlockSpec((BLK, D), lambda i: (i, 0)),
        ],
        out_shape=[
            jax.ShapeDtypeStruct((NPAD, D // 2), jnp.int32),
            jax.ShapeDtypeStruct((NPAD, D), jnp.float32),
        ],
    )(y, sums, g, be, wpe, wpo, bpe, bpo, ws)


def _tc_bn_final_body(y_ref, sums_ref, g_ref, be_ref, o_ref):
    mean, inv = _bn_from_sums(sums_ref[...])
    o_ref[...] = jnp.maximum(
        (y_ref[...] - mean) * (inv * g_ref[...]) + be_ref[...], 0.0)


def _tc_bn_final(y, sums, g, be):
    return pl.pallas_call(
        _tc_bn_final_body,
        grid=(NBLK,),
        in_specs=[
            pl.BlockSpec((BLK, D), lambda i: (i, 0)),
            pl.BlockSpec((8, D), lambda i: (0, 0)),
            pl.BlockSpec((1, D), lambda i: (0, 0)),
            pl.BlockSpec((1, D), lambda i: (0, 0)),
        ],
        out_specs=pl.BlockSpec((BLK, D), lambda i: (i, 0)),
        out_shape=jax.ShapeDtypeStruct((NPAD, D), jnp.float32),
    )(y, sums, g, be)


def _worker(lo_only=False):
    w = lax.axis_index("c") * 16 + lax.axis_index("s")
    lo = w * RPW
    return (lo,) if lo_only else (w, lo)


def _make_partition(epad, cap, npairs):
    nch = 2 * npairs

    @functools.partial(
        pl.kernel,
        out_type=(jax.ShapeDtypeStruct((NW * cap,), jnp.int32),
                  jax.ShapeDtypeStruct((NW * 16,), jnp.int32)),
        mesh=_mesh,
        compiler_params=pltpu.CompilerParams(needs_layout_passes=False),
        scratch_types=[
            pltpu.VMEM((ECHUNK,), jnp.int32),
            pltpu.VMEM((ECHUNK,), jnp.int32),
            pltpu.VMEM((PCAP,), jnp.int32),
            pltpu.VMEM((16,), jnp.int32),
            pltpu.SemaphoreType.DMA,
            pltpu.SemaphoreType.DMA,
        ],
    )
    def part(code_hbm, list_hbm, counts_hbm, buf0, buf1, pend, cbuf,
             sem0, sem1):
        w, lo = _worker()
        lbase = w * cap
        dummy_code = lo + RPW

        pltpu.async_copy(code_hbm.at[pl.ds(0, ECHUNK)], buf0, sem0)
        pltpu.async_copy(code_hbm.at[pl.ds(ECHUNK, ECHUNK)], buf1, sem1)

        def scan_chunk(buf, cnt0):
            def ibody(j, cnt):
                v = buf[pl.ds(j * 16, 16)]
                du = (v & DMASK) - lo
                mask = plsc.bitcast(du, jnp.uint32) < jnp.uint32(RPW)
                csum = plsc.cumsum(mask.astype(jnp.int32))
                plsc.store_scatter(pend, [cnt + csum - 1], v, mask=mask)
                return cnt + csum[15]
            return lax.fori_loop(0, ECHUNK // 16, ibody, cnt0)

        def flush(cnt, optr):
            do = cnt >= ECHUNK

            @pl.when(do)
            def _():
                pltpu.sync_copy(
                    pend.at[pl.ds(0, ECHUNK)],
                    list_hbm.at[pl.ds(pl.multiple_of(lbase + optr, 8),
                                      ECHUNK)])

                def mv(t, _):
                    pend[pl.ds(t * 16, 16)] = pend[pl.ds(ECHUNK + t * 16, 16)]
                    return 0
                lax.fori_loop(0, (PCAP - ECHUNK) // 16, mv, 0)

            cnt = jnp.where(do, cnt - ECHUNK, cnt)
            optr = jnp.where(do, optr + ECHUNK, optr)
            return cnt, optr

        def pbody(p, carry):
            cnt, optr = carry
            pltpu.make_async_copy(code_hbm.at[pl.ds(0, ECHUNK)], buf0,
                                  sem0).wait()
            cnt = scan_chunk(buf0, cnt)
            cnt, optr = flush(cnt, optr)

            @pl.when(2 * p + 2 < nch)
            def _():
                pltpu.async_copy(
                    code_hbm.at[pl.ds((2 * p + 2) * ECHUNK, ECHUNK)],
                    buf0, sem0)

            pltpu.make_async_copy(code_hbm.at[pl.ds(0, ECHUNK)], buf1,
                                  sem1).wait()
            cnt = scan_chunk(buf1, cnt)
            cnt, optr = flush(cnt, optr)

            @pl.when(2 * p + 3 < nch)
            def _():
                pltpu.async_copy(
                    code_hbm.at[pl.ds((2 * p + 3) * ECHUNK, ECHUNK)],
                    buf1, sem1)

            return cnt, optr

        cnt, optr = lax.fori_loop(0, npairs, pbody, (jnp.int32(0),
                                                     jnp.int32(0)))

        dvec = jnp.full((16,), dummy_code, jnp.int32)
        for k in range(8):
            pend[pl.ds(cnt + k * 16, 16)] = dvec
        cnt_pad = ((cnt + 127) // 128) * 128
        pltpu.sync_copy(
            pend.at[pl.ds(0, ECHUNK)],
            list_hbm.at[pl.ds(pl.multiple_of(lbase + optr, 8), ECHUNK)])
        pltpu.sync_copy(
            pend.at[pl.ds(ECHUNK, ECHUNK)],
            list_hbm.at[pl.ds(pl.multiple_of(lbase + optr + ECHUNK, 8),
                              ECHUNK)])
        cbuf[...] = jnp.full((16,), optr + cnt_pad, jnp.int32)
        pltpu.sync_copy(cbuf,
                        counts_hbm.at[pl.ds(pl.multiple_of(w * 16, 8), 16)])

    return part


def _make_segmax(cap):
    @functools.partial(
        pl.kernel,
        out_type=jax.ShapeDtypeStruct((NPAD, D // 2), jnp.int32),
        mesh=_mesh,
        compiler_params=pltpu.CompilerParams(needs_layout_passes=False,
                                             use_tc_tiling_on_sc=False),
        scratch_types=[
            pltpu.VMEM((RPW + 1, D // 2), jnp.int32),
            pltpu.VMEM((GCH,), jnp.int32),
            pltpu.VMEM((GCH,), jnp.int32),
            pltpu.VMEM((GCH,), jnp.int32),
            pltpu.VMEM((GCH,), jnp.int32),
            pltpu.VMEM((GCH,), jnp.int32),
            pltpu.VMEM((GCH,), jnp.int32),
            pltpu.VMEM((GCH, D // 2), jnp.int32),
            pltpu.VMEM((GCH, D // 2), jnp.int32),
            pltpu.VMEM((16,), jnp.int32),
            pltpu.SemaphoreType.DMA,
            pltpu.SemaphoreType.DMA,
            pltpu.SemaphoreType.DMA,
            pltpu.SemaphoreType.DMA,
        ],
    )
    def segmax(m_hbm, list_hbm, counts_hbm, agg_hbm, acc, codesA, codesB,
               sbufA, sbufB, dbufA, dbufB, rowsA, rowsB, cbuf,
               semA, semB, semcA, semcB):
        w, lo = _worker()
        lbase = w * cap
        pltpu.sync_copy(m_hbm.at[pl.ds(pl.multiple_of(lo, 8), RPW)],
                        acc.at[pl.ds(0, RPW)])
        z = jnp.zeros((16,), jnp.int32)
        for c in range(D // 32):
            acc[RPW, pl.ds(c * 16, 16)] = z
        pltpu.sync_copy(counts_hbm.at[pl.ds(pl.multiple_of(w * 16, 8), 16)],
                        cbuf)
        nch = cbuf[...][0] // GCH

        def codes_at(g):
            return list_hbm.at[pl.ds(pl.multiple_of(lbase + g * GCH, 8),
                                     GCH)]

        def decode(codes, sbuf, dbuf):
            for k in range(GCH // 16):
                sl = pl.ds(k * 16, 16)
                v = codes[sl]
                sbuf[sl] = lax.shift_right_logical(v, SHIFT)
                dbuf[sl] = (v & DMASK) - lo

        def accumulate(dbuf, rows):
            def ebody(k, _):
                dvec = dbuf[pl.ds(k * 16, 16)]
                for lane in range(16):
                    d = dvec[lane]
                    i = k * 16 + lane
                    for c in range(D // 32):
                        sl = pl.ds(c * 16, 16)
                        a = plsc.bitcast(acc[d, sl], jnp.bfloat16)
                        r = plsc.bitcast(rows[i, sl], jnp.bfloat16)
                        acc[d, sl] = plsc.bitcast(jnp.maximum(a, r),
                                                  jnp.int32)
                return 0
            lax.fori_loop(0, GCH // 16, ebody, 0)

        @pl.when(nch > 0)
        def _():
            pltpu.sync_copy(codes_at(0), codesA)
            decode(codesA, sbufA, dbufA)
            pltpu.async_copy(m_hbm.at[sbufA], rowsA, semA)

            @pl.when(nch > 1)
            def _():
                pltpu.async_copy(codes_at(1), codesB, semcB)

            def pbody(p, _):
                g = 2 * p

                @pl.when(g + 1 < nch)
                def _():
                    pltpu.make_async_copy(codes_at(1), codesB, semcB).wait()
                    decode(codesB, sbufB, dbufB)
                    pltpu.async_copy(m_hbm.at[sbufB], rowsB, semB)

                @pl.when(g + 2 < nch)
                def _():
                    pltpu.async_copy(codes_at(g + 2), codesA, semcA)

                pltpu.make_async_copy(m_hbm.at[sbufA], rowsA, semA).wait()
                accumulate(dbufA, rowsA)

                @pl.when(g + 2 < nch)
                def _():
                    pltpu.make_async_copy(codes_at(0), codesA, semcA).wait()
                    decode(codesA, sbufA, dbufA)
                    pltpu.async_copy(m_hbm.at[sbufA], rowsA, semA)

                @pl.when(g + 3 < nch)
                def _():
                    pltpu.async_copy(codes_at(g + 3), codesB, semcB)

                @pl.when(g + 1 < nch)
                def _():
                    pltpu.make_async_copy(m_hbm.at[sbufB], rowsB, semB).wait()
                    accumulate(dbufB, rowsB)
                return 0

            lax.fori_loop(0, (nch + 1) // 2, pbody, 0)

        pltpu.sync_copy(acc.at[pl.ds(0, RPW)],
                        agg_hbm.at[pl.ds(pl.multiple_of(lo, 8), RPW)])

    return segmax


def kernel(feat, edge_index, W_pool1, b_pool1, W_self1, W_neigh1, b1,
           gamma1, beta1, W_pool2, b_pool2, W_self2, W_neigh2, b2,
           gamma2, beta2):
    n, d = feat.shape
    assert n == N and d == D
    e = edge_index.shape[1]
    npairs = (e + 2 * ECHUNK - 1) // (2 * ECHUNK)
    epad = npairs * 2 * ECHUNK
    cap = epad + 2 * ECHUNK

    feat_p = jnp.pad(feat, ((0, NPAD - n), (0, 0)))
    code = (edge_index[0].astype(jnp.int32) << SHIFT) \
        | edge_index[1].astype(jnp.int32)
    code_p = jnp.pad(code, (0, epad - e), constant_values=DMASK)

    r = lambda v: v.reshape(1, D)
    rh = lambda v: (v.reshape(1, D)[:, 0::2], v.reshape(1, D)[:, 1::2])

    elist, counts = _make_partition(epad, cap, npairs)(code_p)

    bpe1, bpo1 = rh(b_pool1)
    bpe2, bpo2 = rh(b_pool2)
    m1, s1 = _tc_pre(feat_p, W_pool1[:, 0::2], W_pool1[:, 1::2],
                     bpe1, bpo1, W_self1)
    agg1 = _make_segmax(cap)(m1, elist, counts)
    y1, sums1 = _tc_combine(s1, agg1, W_neigh1[0::2], W_neigh1[1::2], r(b1))
    m2, s2 = _tc_bn_next(y1, sums1, r(gamma1), r(beta1), W_pool2[:, 0::2],
                         W_pool2[:, 1::2], bpe2, bpo2, W_self2)
    agg2 = _make_segmax(cap)(m2, elist, counts)
    y2, sums2 = _tc_combine(s2, agg2, W_neigh2[0::2], W_neigh2[1::2], r(b2))
    out = _tc_bn_final(y2, sums2, r(gamma2), r(beta2))
    return out[:n]
